# Initial kernel scaffold; baseline (speedup 1.0000x reference)
import jax
import jax.numpy as jnp
from jax.experimental import pallas as pl


def kernel(x, edge_index, W, b):
    raise NotImplementedError("write your pallas kernel here")


# speedup vs baseline: 3.2640x; 3.2640x over previous
import functools

import jax
import jax.numpy as jnp
from jax import lax
from jax.experimental import pallas as pl
from jax.experimental.pallas import tpu as pltpu
from jax.experimental.pallas import tpu_sc as plsc

_NCU = 1
_NS = 16
_C = 50
_NB = 16
_ZR = 128
_NPAD = 10240


def _proj_body(x_ref, w1_ref, w2_ref, b_ref, u_ref, v_ref):
    xb = x_ref[...]
    u_ref[...] = jnp.dot(xb, w1_ref[...], preferred_element_type=jnp.float32,
                         precision=lax.Precision.HIGHEST)
    v_ref[...] = jnp.dot(xb, w2_ref[...], preferred_element_type=jnp.float32,
                         precision=lax.Precision.HIGHEST) + b_ref[...]


def _project(x, w1t, w2t, b2d, block_rows=2000):
    n, d = x.shape
    dout = w1t.shape[1]
    return pl.pallas_call(
        _proj_body,
        grid=(n // block_rows,),
        in_specs=[
            pl.BlockSpec((block_rows, d), lambda i: (i, 0)),
            pl.BlockSpec((d, dout), lambda i: (0, 0)),
            pl.BlockSpec((d, dout), lambda i: (0, 0)),
            pl.BlockSpec((1, dout), lambda i: (0, 0)),
        ],
        out_specs=[
            pl.BlockSpec((block_rows, dout), lambda i: (i, 0)),
            pl.BlockSpec((block_rows, dout), lambda i: (i, 0)),
        ],
        out_shape=[
            jax.ShapeDtypeStruct((n, dout), jnp.float32),
            jax.ShapeDtypeStruct((n, dout), jnp.float32),
        ],
    )(x, w1t, w2t, b2d)


def _final_body(u_ref, agg_ref, w2_ref, b_ref, o_ref):
    z = jnp.dot(agg_ref[...], w2_ref[...], preferred_element_type=jnp.float32,
                precision=lax.Precision.HIGHEST)
    o_ref[...] = jnp.maximum(z + u_ref[...] + b_ref[...], 0.0)


def _final(u, agg, w2t, b2d, block_rows=2000):
    n, dout = u.shape
    return pl.pallas_call(
        _final_body,
        grid=(n // block_rows,),
        in_specs=[
            pl.BlockSpec((block_rows, dout), lambda i: (i, 0)),
            pl.BlockSpec((block_rows, dout), lambda i: (i, 0)),
            pl.BlockSpec((dout, dout), lambda i: (0, 0)),
            pl.BlockSpec((1, dout), lambda i: (0, 0)),
        ],
        out_specs=pl.BlockSpec((block_rows, dout), lambda i: (i, 0)),
        out_shape=jax.ShapeDtypeStruct((n, dout), jnp.float32),
    )(u, agg, w2t, b2d)


def _edge_agg(u, v, rows3, cols3):
    n, d = u.shape
    nw = _NCU * _NS
    nch = rows3.shape[1]
    nblk = nch // _NB
    rows_tile = _NPAD // _NS
    nz = rows_tile // _ZR

    mesh = plsc.VectorSubcoreMesh(core_axis_name="c", subcore_axis_name="s",
                                  num_cores=_NCU)

    @functools.partial(
        pl.kernel,
        out_type=jax.ShapeDtypeStruct((_NPAD, d), jnp.float32),
        mesh=mesh,
        scratch_types=[
            pltpu.VMEM((_NB, _C), jnp.int32),
            pltpu.VMEM((_NB, _C), jnp.int32),
            pltpu.VMEM((_C, d), jnp.float32),
            pltpu.VMEM((_C, d), jnp.float32),
            pltpu.VMEM((_C, d), jnp.float32),
            pltpu.VMEM_SHARED((_NPAD, d), jnp.float32),
            pltpu.SemaphoreType.DMA,
        ],
    )
    def k(u_hbm, v_hbm, r3_hbm, c3_hbm, out_hbm,
          rows_ib, cols_ib, ubuf, vbuf, mbuf, agg_sh, sem_g):
        s = lax.axis_index("s")

        @pl.loop(0, _C)
        def _(i):
            for g in range(d // 16):
                mbuf[i, pl.ds(g * 16, 16)] = jnp.zeros((16,), jnp.float32)

        rbase = s * rows_tile
        zc = rows_tile // _C
        for z in range(zc):
            pltpu.sync_copy(mbuf, agg_sh.at[pl.ds(rbase + z * _C, _C)])
        rem = rows_tile - zc * _C
        if rem:
            pltpu.sync_copy(mbuf.at[pl.ds(0, rem)],
                            agg_sh.at[pl.ds(rbase + zc * _C, rem)])
        plsc.subcore_barrier()

        @pl.loop(0, nblk)
        def _(k_):
            pltpu.sync_copy(r3_hbm.at[s, pl.ds(k_ * _NB, _NB)], rows_ib)
            pltpu.sync_copy(c3_hbm.at[s, pl.ds(k_ * _NB, _NB)], cols_ib)

            @pl.loop(0, _NB)
            def _(t):
                pltpu.async_copy(u_hbm.at[rows_ib.at[t]], ubuf, sem_g)
                pltpu.async_copy(v_hbm.at[cols_ib.at[t]], vbuf, sem_g)
                pltpu.make_async_copy(u_hbm.at[rows_ib.at[t]], ubuf, sem_g).wait()
                pltpu.make_async_copy(v_hbm.at[cols_ib.at[t]], vbuf, sem_g).wait()

                @pl.loop(0, _C)
                def _(r):
                    for g in range(d // 16):
                        sl = pl.ds(g * 16, 16)
                        mbuf[r, sl] = jnp.maximum(ubuf[r, sl] + vbuf[r, sl], 0.0)

                pltpu.sync_copy(mbuf, agg_sh.at[rows_ib.at[t]], add=True)

        plsc.subcore_barrier()

        for z in range(nz):
            r0 = rbase + z * _ZR
            pltpu.sync_copy(agg_sh.at[pl.ds(r0, _ZR)], out_hbm.at[pl.ds(r0, _ZR)])

    return k(u, v, rows3, cols3)


def kernel(x, edge_index, W, b):
    n, d = x.shape
    dout = W.shape[0]
    w1t = jnp.transpose(W[:, :d])
    w2t = jnp.transpose(W[:, d:])
    b2d = b.reshape(1, dout)

    rows = edge_index[0]
    cols = edge_index[1]
    e = rows.shape[0]
    nw = _NCU * _NS
    rows3 = rows.reshape(nw, e // (nw * _C), _C)
    cols3 = cols.reshape(nw, e // (nw * _C), _C)

    u, v = _project(x, w1t, w2t, b2d)
    agg = _edge_agg(u, v, rows3, cols3)
    return _final(u, agg, w2t, b2d)


# speedup vs baseline: 4.7423x; 1.4529x over previous
import functools

import jax
import jax.numpy as jnp
from jax import lax
from jax.experimental import pallas as pl
from jax.experimental.pallas import tpu as pltpu
from jax.experimental.pallas import tpu_sc as plsc

_NCU = 1
_NS = 16
_C = 50
_NB = 16
_ZR = 128
_NPAD = 10240


def _proj_body(x_ref, w1_ref, w2_ref, b_ref, u_ref, v_ref):
    xb = x_ref[...]
    u_ref[...] = jnp.dot(xb, w1_ref[...], preferred_element_type=jnp.float32,
                         precision=lax.Precision.HIGHEST)
    v_ref[...] = jnp.dot(xb, w2_ref[...], preferred_element_type=jnp.float32,
                         precision=lax.Precision.HIGHEST) + b_ref[...]


def _project(x, w1t, w2t, b2d, block_rows=2000):
    n, d = x.shape
    dout = w1t.shape[1]
    return pl.pallas_call(
        _proj_body,
        grid=(n // block_rows,),
        in_specs=[
            pl.BlockSpec((block_rows, d), lambda i: (i, 0)),
            pl.BlockSpec((d, dout), lambda i: (0, 0)),
            pl.BlockSpec((d, dout), lambda i: (0, 0)),
            pl.BlockSpec((1, dout), lambda i: (0, 0)),
        ],
        out_specs=[
            pl.BlockSpec((block_rows, dout), lambda i: (i, 0)),
            pl.BlockSpec((block_rows, dout), lambda i: (i, 0)),
        ],
        out_shape=[
            jax.ShapeDtypeStruct((n, dout), jnp.float32),
            jax.ShapeDtypeStruct((n, dout), jnp.float32),
        ],
    )(x, w1t, w2t, b2d)


def _final_body(u_ref, agg_ref, w2_ref, b_ref, o_ref):
    z = jnp.dot(agg_ref[...], w2_ref[...], preferred_element_type=jnp.float32,
                precision=lax.Precision.HIGHEST)
    o_ref[...] = jnp.maximum(z + u_ref[...] + b_ref[...], 0.0)


def _final(u, agg, w2t, b2d, block_rows=2000):
    n, dout = u.shape
    return pl.pallas_call(
        _final_body,
        grid=(n // block_rows,),
        in_specs=[
            pl.BlockSpec((block_rows, dout), lambda i: (i, 0)),
            pl.BlockSpec((block_rows, dout), lambda i: (i, 0)),
            pl.BlockSpec((dout, dout), lambda i: (0, 0)),
            pl.BlockSpec((1, dout), lambda i: (0, 0)),
        ],
        out_specs=pl.BlockSpec((block_rows, dout), lambda i: (i, 0)),
        out_shape=jax.ShapeDtypeStruct((n, dout), jnp.float32),
    )(u, agg, w2t, b2d)


def _edge_agg(u, v, rows3, cols3):
    n, d = u.shape
    nw = _NCU * _NS
    nch = rows3.shape[1]
    nblk = nch // _NB
    rows_tile = _NPAD // _NS
    nz = rows_tile // _ZR

    mesh = plsc.VectorSubcoreMesh(core_axis_name="c", subcore_axis_name="s",
                                  num_cores=_NCU)

    @functools.partial(
        pl.kernel,
        out_type=jax.ShapeDtypeStruct((_NPAD, d), jnp.float32),
        mesh=mesh,
        scratch_types=[
            pltpu.VMEM((_NB, _C), jnp.int32),
            pltpu.VMEM((_NB, _C), jnp.int32),
            pltpu.VMEM((_C, d), jnp.float32),
            pltpu.VMEM((_C, d), jnp.float32),
            pltpu.VMEM((_C, d), jnp.float32),
            pltpu.VMEM((_C, d), jnp.float32),
            pltpu.VMEM_SHARED((_NPAD, d), jnp.float32),
            pltpu.SemaphoreType.DMA,
            pltpu.SemaphoreType.DMA,
        ],
    )
    def k(u_hbm, v_hbm, r3_hbm, c3_hbm, out_hbm,
          rows_ib, cols_ib, ubuf0, ubuf1, vbuf0, vbuf1, agg_sh, sem_g, sem_s):
        s = lax.axis_index("s")
        ub = (ubuf0, ubuf1)
        vb = (vbuf0, vbuf1)

        @pl.loop(0, _C)
        def _(i):
            for g in range(d // 16):
                ubuf0[i, pl.ds(g * 16, 16)] = jnp.zeros((16,), jnp.float32)

        rbase = s * rows_tile
        zc = rows_tile // _C
        for z in range(zc):
            pltpu.sync_copy(ubuf0, agg_sh.at[pl.ds(rbase + z * _C, _C)])
        rem = rows_tile - zc * _C
        if rem:
            pltpu.sync_copy(ubuf0.at[pl.ds(0, rem)],
                            agg_sh.at[pl.ds(rbase + zc * _C, rem)])
        plsc.subcore_barrier()

        @pl.loop(0, nblk)
        def _(k_):
            pltpu.sync_copy(r3_hbm.at[s, pl.ds(k_ * _NB, _NB)], rows_ib)
            pltpu.sync_copy(c3_hbm.at[s, pl.ds(k_ * _NB, _NB)], cols_ib)

            pltpu.async_copy(u_hbm.at[rows_ib.at[0]], ubuf0, sem_g)
            pltpu.async_copy(v_hbm.at[cols_ib.at[0]], vbuf0, sem_g)

            @pl.loop(0, _NB, step=2)
            def _(tt):
                for b in (0, 1):
                    t = tt + b
                    pltpu.make_async_copy(u_hbm.at[rows_ib.at[t]], ub[b],
                                          sem_g).wait()
                    pltpu.make_async_copy(v_hbm.at[cols_ib.at[t]], vb[b],
                                          sem_g).wait()

                    def _wait_prev_scatter():
                        pltpu.make_async_copy(
                            vb[1 - b], agg_sh.at[rows_ib.at[0]], sem_s).wait()
                    if b == 1:
                        _wait_prev_scatter()
                    else:
                        pl.when(tt >= 1)(_wait_prev_scatter)

                    def _issue_next_gathers():
                        pltpu.async_copy(u_hbm.at[rows_ib.at[t + 1]], ub[1 - b],
                                         sem_g)
                        pltpu.async_copy(v_hbm.at[cols_ib.at[t + 1]], vb[1 - b],
                                         sem_g)
                    if b == 0:
                        _issue_next_gathers()
                    else:
                        pl.when(tt < _NB - 2)(_issue_next_gathers)

                    @pl.loop(0, _C)
                    def _(r):
                        for g in range(d // 16):
                            sl = pl.ds(g * 16, 16)
                            vb[b][r, sl] = jnp.maximum(
                                ub[b][r, sl] + vb[b][r, sl], 0.0)

                    pltpu.async_copy(vb[b], agg_sh.at[rows_ib.at[t]], sem_s,
                                     add=True)

            pltpu.make_async_copy(vbuf1, agg_sh.at[rows_ib.at[0]], sem_s).wait()

        plsc.subcore_barrier()

        for z in range(nz):
            r0 = rbase + z * _ZR
            pltpu.sync_copy(agg_sh.at[pl.ds(r0, _ZR)], out_hbm.at[pl.ds(r0, _ZR)])

    return k(u, v, rows3, cols3)


def kernel(x, edge_index, W, b):
    n, d = x.shape
    dout = W.shape[0]
    w1t = jnp.transpose(W[:, :d])
    w2t = jnp.transpose(W[:, d:])
    b2d = b.reshape(1, dout)

    rows = edge_index[0]
    cols = edge_index[1]
    e = rows.shape[0]
    nw = _NCU * _NS
    rows3 = rows.reshape(nw, e // (nw * _C), _C)
    cols3 = cols.reshape(nw, e // (nw * _C), _C)

    u, v = _project(x, w1t, w2t, b2d)
    agg = _edge_agg(u, v, rows3, cols3)
    return _final(u, agg, w2t, b2d)
